# baseline (speedup 1.0000x reference)
import functools

import jax
import jax.numpy as jnp
from jax import lax
from jax.experimental import pallas as pl
from jax.experimental.pallas import tpu as pltpu
from jax.experimental.pallas import tpu_sc as plsc

N = 10000
D = 128
E = 320000
NPAD = 10240
EALL = E + N
CH = 128
CPT = 82
NCHUNK = 32 * CPT
EPAD = NCHUNK * CH
RPT = NPAD // 16
BLK = 1024
GRID = NPAD // BLK

_f32 = jnp.float32


def _bcast_lane(v, m):
    idx = jnp.full((16, 1), m, jnp.int32)
    dn = lax.GatherDimensionNumbers(
        offset_dims=(), collapsed_slice_dims=(0,), start_index_map=(0,))
    return lax.gather(v, idx, dn, (1,),
                      mode=lax.GatherScatterMode.PROMISE_IN_BOUNDS)


def _tc_head_body(x_ref, w_ref, a_ref, h_ref, p_ref, mx_ref):
    h = jnp.dot(x_ref[...], w_ref[...], preferred_element_type=_f32)
    p = jnp.dot(h, a_ref[...], preferred_element_type=_f32)
    h_ref[...] = h
    p_ref[...] = p
    mx_ref[pl.ds(pl.program_id(0), 1), :] = jnp.max(p, axis=0, keepdims=True)


_tc_head = pl.pallas_call(
    _tc_head_body,
    grid=(GRID,),
    in_specs=[
        pl.BlockSpec((BLK, D), lambda i: (i, 0)),
        pl.BlockSpec((D, D), lambda i: (0, 0)),
        pl.BlockSpec((D, 8), lambda i: (0, 0)),
    ],
    out_specs=[
        pl.BlockSpec((BLK, D), lambda i: (i, 0)),
        pl.BlockSpec((BLK, 8), lambda i: (i, 0)),
        pl.BlockSpec((GRID, 8), lambda i: (0, 0)),
    ],
    out_shape=[
        jax.ShapeDtypeStruct((NPAD, D), _f32),
        jax.ShapeDtypeStruct((NPAD, 8), _f32),
        jax.ShapeDtypeStruct((GRID, 8), _f32),
    ],
)


def _tc_mid_body(p0_ref, p1_ref, d0_ref, d1_ref, b_ref, w_ref, a_ref,
                 h_ref, p_ref, mx_ref):
    den = d0_ref[...] + d1_ref[...] + 1e-16
    x = jnp.maximum((p0_ref[...] + p1_ref[...]) / den + b_ref[...], 0.0)
    h = jnp.dot(x, w_ref[...], preferred_element_type=_f32)
    p = jnp.dot(h, a_ref[...], preferred_element_type=_f32)
    h_ref[...] = h
    p_ref[...] = p
    mx_ref[pl.ds(pl.program_id(0), 1), :] = jnp.max(p, axis=0, keepdims=True)


_tc_mid = pl.pallas_call(
    _tc_mid_body,
    grid=(GRID,),
    in_specs=[
        pl.BlockSpec((BLK, D), lambda i: (i, 0)),
        pl.BlockSpec((BLK, D), lambda i: (i, 0)),
        pl.BlockSpec((BLK, 1), lambda i: (i, 0)),
        pl.BlockSpec((BLK, 1), lambda i: (i, 0)),
        pl.BlockSpec((1, D), lambda i: (0, 0)),
        pl.BlockSpec((D, D), lambda i: (0, 0)),
        pl.BlockSpec((D, 8), lambda i: (0, 0)),
    ],
    out_specs=[
        pl.BlockSpec((BLK, D), lambda i: (i, 0)),
        pl.BlockSpec((BLK, 8), lambda i: (i, 0)),
        pl.BlockSpec((GRID, 8), lambda i: (0, 0)),
    ],
    out_shape=[
        jax.ShapeDtypeStruct((NPAD, D), _f32),
        jax.ShapeDtypeStruct((NPAD, 8), _f32),
        jax.ShapeDtypeStruct((GRID, 8), _f32),
    ],
)


def _tc_tail_body(p0_ref, p1_ref, d0_ref, d1_ref, b_ref, o_ref):
    den = d0_ref[...] + d1_ref[...] + 1e-16
    o_ref[...] = (p0_ref[...] + p1_ref[...]) / den + b_ref[...]


_tc_tail = pl.pallas_call(
    _tc_tail_body,
    grid=(GRID,),
    in_specs=[
        pl.BlockSpec((BLK, D), lambda i: (i, 0)),
        pl.BlockSpec((BLK, D), lambda i: (i, 0)),
        pl.BlockSpec((BLK, 1), lambda i: (i, 0)),
        pl.BlockSpec((BLK, 1), lambda i: (i, 0)),
        pl.BlockSpec((1, D), lambda i: (0, 0)),
    ],
    out_specs=pl.BlockSpec((BLK, D), lambda i: (i, 0)),
    out_shape=jax.ShapeDtypeStruct((NPAD, D), _f32),
)


@functools.partial(
    pl.kernel,
    mesh=plsc.VectorSubcoreMesh(core_axis_name="c", subcore_axis_name="s"),
    compiler_params=pltpu.CompilerParams(needs_layout_passes=False),
    out_type=[
        jax.ShapeDtypeStruct((2 * NPAD, D), _f32),
        jax.ShapeDtypeStruct((2 * NPAD,), _f32),
    ],
    scratch_types=[
        pltpu.VMEM((NPAD,), jnp.int32),
        pltpu.VMEM((8, CH), jnp.int32),
        pltpu.VMEM((8, CH), jnp.int32),
        pltpu.VMEM((CH,), _f32),
        pltpu.VMEM((CH, D), _f32),
        pltpu.VMEM((CH,), _f32),
        pltpu.VMEM((CH, D), _f32),
        pltpu.VMEM((16,), _f32),
        pltpu.VMEM((RPT,), _f32),
        pltpu.VMEM_SHARED((NPAD, D), _f32),
        pltpu.VMEM_SHARED((NPAD,), _f32),
        pltpu.SemaphoreType.DMA,
        pltpu.SemaphoreType.DMA,
        pltpu.SemaphoreType.DMA,
        pltpu.SemaphoreType.DMA,
        pltpu.SemaphoreType.DMA,
        pltpu.SemaphoreType.DMA,
    ],
)
def _sc_edge(h_hbm, sd_hbm, pal_hbm, g_hbm,
             outp_hbm, denp_hbm,
             pal_v, idx_a, idx_b,
             expa_a, rows_a, expa_b, rows_b,
             g_v, dbuf, out_sp, den_sp,
             gsem_a, gsem_b, ssem_a, ssem_b, psem_a, psem_b):
    cid = lax.axis_index("c")
    sid = lax.axis_index("s")
    wid = sid * 2 + cid
    tbase = sid * RPT

    bufs = ((idx_a, expa_a, rows_a, gsem_a, ssem_a, psem_a),
            (idx_b, expa_b, rows_b, gsem_b, ssem_b, psem_b))

    def _f16_bits_to_f32(v):
        sgn = lax.shift_left(v & 0x8000, 16)
        rest = lax.shift_left(v & 0x7FFF, 13) + 0x38000000
        return lax.bitcast_convert_type(sgn | rest, _f32)

    def start_packed(b, j):
        idx, _, _, _, _, psem = bufs[b]
        pltpu.async_copy(sd_hbm.at[wid, j], idx.at[2], psem)

    def wait_packed(b, j):
        idx, _, _, _, _, psem = bufs[b]
        pltpu.make_async_copy(sd_hbm.at[wid, j], idx.at[2], psem).wait()

    def unpack_idx(b):
        idx = bufs[b][0]

        def _u(k, carry):
            sl = pl.ds(k * 16, 16)
            v = idx[2, sl]
            idx[0, sl] = v & 0xFFFF
            idx[1, sl] = lax.shift_right_logical(v, 16)
            return carry

        lax.fori_loop(0, CH // 16, _u, 0)

    def start_gathers(b):
        idx, _, rows, gsem, _, _ = bufs[b]
        pltpu.async_copy(h_hbm.at[idx.at[0]], rows, gsem)

    def wait_gathers(b):
        idx, _, rows, gsem, _, _ = bufs[b]
        pltpu.make_async_copy(h_hbm.at[idx.at[0]], rows, gsem).wait()

    def start_scatters(b):
        idx, expa, rows, _, ssem, _ = bufs[b]
        pltpu.async_copy(expa, den_sp.at[idx.at[1]], ssem, add=True)
        pltpu.async_copy(rows, out_sp.at[idx.at[1]], ssem, add=True)

    def wait_scatters(b):
        idx, expa, rows, _, ssem, _ = bufs[b]
        pltpu.make_async_copy(expa, den_sp.at[idx.at[1]], ssem).wait()
        pltpu.make_async_copy(rows, out_sp.at[idx.at[1]], ssem).wait()

    def compute(b):
        idx, expa, rows, _, _, _ = bufs[b]

        @plsc.parallel_loop(0, CH // 16, 1, unroll=4)
        def _grp(k):
            sl16 = pl.ds(k * 16, 16)
            s16 = idx[0, sl16]
            d16 = idx[1, sl16]
            ps = plsc.load_gather(pal_v, [s16])
            pd = plsc.load_gather(pal_v, [d16])
            a = (_f16_bits_to_f32(lax.shift_right_logical(ps, 16))
                 + _f16_bits_to_f32(pd & 0xFFFF))
            a = jnp.maximum(a, 0.2 * a)
            ex = jnp.exp(a - g_v[...])
            expa[sl16] = ex
            for m in range(16):
                coef = _bcast_lane(ex, m)
                e = k * 16 + m
                for q in range(D // 16):
                    sl = pl.ds(q * 16, 16)
                    rows[e, sl] = rows[e, sl] * coef

    pltpu.sync_copy(pal_hbm, pal_v)
    pltpu.sync_copy(g_hbm, g_v)
    start_packed(0, 0)
    start_packed(1, 1)
    wait_packed(0, 0)
    unpack_idx(0)
    start_packed(0, 2)
    start_gathers(0)

    def _zrow(e, carry):
        for q in range(D // 16):
            rows_b[e, pl.ds(q * 16, 16)] = jnp.zeros((16,), _f32)
        return carry

    lax.fori_loop(0, CH, _zrow, 0)

    def _zd(i, carry):
        dbuf[pl.ds(i * 16, 16)] = jnp.zeros((16,), _f32)
        return carry

    lax.fori_loop(0, RPT // 16, _zd, 0)

    for k in range(RPT // CH):
        pltpu.sync_copy(rows_b, out_sp.at[pl.ds(tbase + k * CH, CH)])
    pltpu.sync_copy(dbuf, den_sp.at[pl.ds(tbase, RPT)])
    plsc.subcore_barrier()

    NP = CPT // 2

    def _pair(p, carry):
        ja = 2 * p
        jb = ja + 1

        @pl.when(p > 0)
        def _():
            wait_scatters(1)

        wait_packed(1, jb)
        unpack_idx(1)
        start_packed(1, jnp.minimum(jb + 2, CPT - 1))
        start_gathers(1)
        wait_gathers(0)
        compute(0)
        start_scatters(0)

        wait_gathers(1)
        compute(1)
        start_scatters(1)

        @pl.when(p < NP - 1)
        def _():
            wait_scatters(0)
            wait_packed(0, ja + 2)
            unpack_idx(0)
            start_packed(0, jnp.minimum(ja + 4, CPT - 1))
            start_gathers(0)

        return carry

    lax.fori_loop(0, NP, _pair, 0)
    wait_scatters(0)
    wait_scatters(1)
    wait_packed(0, CPT - 1)
    wait_packed(1, CPT - 1)
    plsc.subcore_barrier()

    obase = cid * NPAD
    for k in range(RPT // CH):
        pltpu.sync_copy(out_sp.at[pl.ds(tbase + k * CH, CH)], rows_a)
        pltpu.sync_copy(rows_a, outp_hbm.at[pl.ds(obase + tbase + k * CH, CH)])
    pltpu.sync_copy(den_sp.at[pl.ds(tbase, RPT)], dbuf)
    pltpu.sync_copy(dbuf, denp_hbm.at[pl.ds(obase + tbase, RPT)])


def _avec(a_s, a_d):
    A = jnp.zeros((D, 8), _f32)
    return A.at[:, 0].set(a_s).at[:, 1].set(a_d)


def _pal(P):
    hi = lax.bitcast_convert_type(P[:, 0].astype(jnp.float16), jnp.uint16)
    lo = lax.bitcast_convert_type(P[:, 1].astype(jnp.float16), jnp.uint16)
    return (hi.astype(jnp.int32) << 16) | lo.astype(jnp.int32)


def _gvec(mx):
    g = jnp.maximum(jnp.max(mx[:, 0]) + jnp.max(mx[:, 1]), 0.0)
    return jnp.full((16,), g, _f32)


def kernel(x, edge_index, edge_weight, W1, a_src1, a_dst1, b1,
           W2, a_src2, a_dst2, b2, W3, a_src3, a_dst3, b3):
    del edge_weight
    xp = jnp.zeros((NPAD, D), _f32).at[:N].set(x)
    loop = jnp.arange(N, dtype=jnp.int32)
    src = jnp.concatenate([edge_index[0].astype(jnp.int32), loop])
    dst = jnp.concatenate([edge_index[1].astype(jnp.int32), loop])
    padidx = N + (jnp.arange(EPAD - EALL, dtype=jnp.int32) % (NPAD - N))
    srcf = jnp.concatenate([src, padidx])
    dstf = jnp.concatenate([dst, padidx])
    sd2 = (srcf | (dstf << 16)).reshape(32, CPT, CH)

    h, P, mx = _tc_head(xp, W1, _avec(a_src1, a_dst1))
    outp, denp = _sc_edge(h, sd2, _pal(P), _gvec(mx))

    h, P, mx = _tc_mid(outp[:NPAD], outp[NPAD:], denp[:NPAD, None],
                       denp[NPAD:, None], b1.reshape(1, D), W2,
                       _avec(a_src2, a_dst2))
    outp, denp = _sc_edge(h, sd2, _pal(P), _gvec(mx))

    h, P, mx = _tc_mid(outp[:NPAD], outp[NPAD:], denp[:NPAD, None],
                       denp[NPAD:, None], b2.reshape(1, D), W3,
                       _avec(a_src3, a_dst3))
    outp, denp = _sc_edge(h, sd2, _pal(P), _gvec(mx))

    out = _tc_tail(outp[:NPAD], outp[NPAD:], denp[:NPAD, None],
                   denp[NPAD:, None], b3.reshape(1, D))
    return out[:N]


# speedup vs baseline: 1.0224x; 1.0224x over previous
import functools

import jax
import jax.numpy as jnp
from jax import lax
from jax.experimental import pallas as pl
from jax.experimental.pallas import tpu as pltpu
from jax.experimental.pallas import tpu_sc as plsc

N = 10000
D = 128
E = 320000
NPAD = 10240
EALL = E + N
CH = 128
CPT = 82
NCHUNK = 32 * CPT
EPAD = NCHUNK * CH
RPT = NPAD // 16
BLK = 1024
GRID = NPAD // BLK

_f32 = jnp.float32


def _bcast_lane(v, m):
    idx = jnp.full((16, 1), m, jnp.int32)
    dn = lax.GatherDimensionNumbers(
        offset_dims=(), collapsed_slice_dims=(0,), start_index_map=(0,))
    return lax.gather(v, idx, dn, (1,),
                      mode=lax.GatherScatterMode.PROMISE_IN_BOUNDS)


def _tc_head_body(x_ref, w_ref, a_ref, h_ref, p_ref, mx_ref):
    h = jnp.dot(x_ref[...], w_ref[...], preferred_element_type=_f32)
    p = jnp.dot(h, a_ref[...], preferred_element_type=_f32)
    h_ref[...] = h
    p_ref[...] = p
    mx_ref[pl.ds(pl.program_id(0), 1), :] = jnp.max(p, axis=0, keepdims=True)


_tc_head = pl.pallas_call(
    _tc_head_body,
    grid=(GRID,),
    in_specs=[
        pl.BlockSpec((BLK, D), lambda i: (i, 0)),
        pl.BlockSpec((D, D), lambda i: (0, 0)),
        pl.BlockSpec((D, 8), lambda i: (0, 0)),
    ],
    out_specs=[
        pl.BlockSpec((BLK, D), lambda i: (i, 0)),
        pl.BlockSpec((BLK, 8), lambda i: (i, 0)),
        pl.BlockSpec((GRID, 8), lambda i: (0, 0)),
    ],
    out_shape=[
        jax.ShapeDtypeStruct((NPAD, D), _f32),
        jax.ShapeDtypeStruct((NPAD, 8), _f32),
        jax.ShapeDtypeStruct((GRID, 8), _f32),
    ],
)


def _tc_mid_body(p0_ref, p1_ref, d0_ref, d1_ref, b_ref, w_ref, a_ref,
                 h_ref, p_ref, mx_ref):
    den = d0_ref[...] + d1_ref[...] + 1e-16
    x = jnp.maximum((p0_ref[...] + p1_ref[...]) / den + b_ref[...], 0.0)
    h = jnp.dot(x, w_ref[...], preferred_element_type=_f32)
    p = jnp.dot(h, a_ref[...], preferred_element_type=_f32)
    h_ref[...] = h
    p_ref[...] = p
    mx_ref[pl.ds(pl.program_id(0), 1), :] = jnp.max(p, axis=0, keepdims=True)


_tc_mid = pl.pallas_call(
    _tc_mid_body,
    grid=(GRID,),
    in_specs=[
        pl.BlockSpec((BLK, D), lambda i: (i, 0)),
        pl.BlockSpec((BLK, D), lambda i: (i, 0)),
        pl.BlockSpec((BLK, 1), lambda i: (i, 0)),
        pl.BlockSpec((BLK, 1), lambda i: (i, 0)),
        pl.BlockSpec((1, D), lambda i: (0, 0)),
        pl.BlockSpec((D, D), lambda i: (0, 0)),
        pl.BlockSpec((D, 8), lambda i: (0, 0)),
    ],
    out_specs=[
        pl.BlockSpec((BLK, D), lambda i: (i, 0)),
        pl.BlockSpec((BLK, 8), lambda i: (i, 0)),
        pl.BlockSpec((GRID, 8), lambda i: (0, 0)),
    ],
    out_shape=[
        jax.ShapeDtypeStruct((NPAD, D), _f32),
        jax.ShapeDtypeStruct((NPAD, 8), _f32),
        jax.ShapeDtypeStruct((GRID, 8), _f32),
    ],
)


def _tc_tail_body(p0_ref, p1_ref, d0_ref, d1_ref, b_ref, o_ref):
    den = d0_ref[...] + d1_ref[...] + 1e-16
    o_ref[...] = (p0_ref[...] + p1_ref[...]) / den + b_ref[...]


_tc_tail = pl.pallas_call(
    _tc_tail_body,
    grid=(GRID,),
    in_specs=[
        pl.BlockSpec((BLK, D), lambda i: (i, 0)),
        pl.BlockSpec((BLK, D), lambda i: (i, 0)),
        pl.BlockSpec((BLK, 1), lambda i: (i, 0)),
        pl.BlockSpec((BLK, 1), lambda i: (i, 0)),
        pl.BlockSpec((1, D), lambda i: (0, 0)),
    ],
    out_specs=pl.BlockSpec((BLK, D), lambda i: (i, 0)),
    out_shape=jax.ShapeDtypeStruct((NPAD, D), _f32),
)


@functools.partial(
    pl.kernel,
    mesh=plsc.VectorSubcoreMesh(core_axis_name="c", subcore_axis_name="s"),
    compiler_params=pltpu.CompilerParams(needs_layout_passes=False),
    out_type=[
        jax.ShapeDtypeStruct((2 * NPAD, D), _f32),
        jax.ShapeDtypeStruct((2 * NPAD,), _f32),
    ],
    scratch_types=[
        pltpu.VMEM((NPAD,), jnp.int32),
        pltpu.VMEM((8, CH), jnp.int32),
        pltpu.VMEM((8, CH), jnp.int32),
        pltpu.VMEM((CH,), _f32),
        pltpu.VMEM((CH, D), _f32),
        pltpu.VMEM((CH,), _f32),
        pltpu.VMEM((CH, D), _f32),
        pltpu.VMEM((16,), _f32),
        pltpu.VMEM((RPT,), _f32),
        pltpu.VMEM_SHARED((NPAD, D), _f32),
        pltpu.VMEM_SHARED((NPAD,), _f32),
        pltpu.SemaphoreType.DMA,
        pltpu.SemaphoreType.DMA,
        pltpu.SemaphoreType.DMA,
        pltpu.SemaphoreType.DMA,
        pltpu.SemaphoreType.DMA,
        pltpu.SemaphoreType.DMA,
    ],
)
def _sc_edge(h_hbm, sd_hbm, pal_hbm, g_hbm,
             outp_hbm, denp_hbm,
             pal_v, idx_a, idx_b,
             expa_a, rows_a, expa_b, rows_b,
             g_v, dbuf, out_sp, den_sp,
             gsem_a, gsem_b, ssem_a, ssem_b, psem_a, psem_b):
    cid = lax.axis_index("c")
    sid = lax.axis_index("s")
    wid = sid * 2 + cid
    tbase = sid * RPT

    bufs = ((idx_a, expa_a, rows_a, gsem_a, ssem_a, psem_a),
            (idx_b, expa_b, rows_b, gsem_b, ssem_b, psem_b))

    def _f16_bits_to_f32(v):
        sgn = lax.shift_left(v & 0x8000, 16)
        rest = lax.shift_left(v & 0x7FFF, 13) + 0x38000000
        return lax.bitcast_convert_type(sgn | rest, _f32)

    def start_packed(b, j):
        idx, _, _, _, _, psem = bufs[b]
        pltpu.async_copy(sd_hbm.at[wid, j], idx.at[2], psem)

    def wait_packed(b, j):
        idx, _, _, _, _, psem = bufs[b]
        pltpu.make_async_copy(sd_hbm.at[wid, j], idx.at[2], psem).wait()

    def unpack_idx(b):
        idx = bufs[b][0]

        def _u(k, carry):
            sl = pl.ds(k * 16, 16)
            v = idx[2, sl]
            idx[0, sl] = v & 0xFFFF
            idx[1, sl] = lax.shift_right_logical(v, 16)
            return carry

        lax.fori_loop(0, CH // 16, _u, 0)

    def start_gathers(b):
        idx, _, rows, gsem, _, _ = bufs[b]
        pltpu.async_copy(h_hbm.at[idx.at[0]], rows, gsem)

    def wait_gathers(b):
        idx, _, rows, gsem, _, _ = bufs[b]
        pltpu.make_async_copy(h_hbm.at[idx.at[0]], rows, gsem).wait()

    def start_scatters(b):
        idx, expa, rows, _, ssem, _ = bufs[b]
        pltpu.async_copy(expa, den_sp.at[idx.at[1]], ssem, add=True)
        pltpu.async_copy(rows, out_sp.at[idx.at[1]], ssem, add=True)

    def wait_scatters(b):
        idx, expa, rows, _, ssem, _ = bufs[b]
        pltpu.make_async_copy(expa, den_sp.at[idx.at[1]], ssem).wait()
        pltpu.make_async_copy(rows, out_sp.at[idx.at[1]], ssem).wait()

    def compute(b):
        idx, expa, rows, _, _, _ = bufs[b]

        @plsc.parallel_loop(0, CH // 16, 1, unroll=2)
        def _grp(k):
            sl16 = pl.ds(k * 16, 16)
            s16 = idx[0, sl16]
            d16 = idx[1, sl16]
            ps = plsc.load_gather(pal_v, [s16])
            pd = plsc.load_gather(pal_v, [d16])
            a = (_f16_bits_to_f32(lax.shift_right_logical(ps, 16))
                 + _f16_bits_to_f32(pd & 0xFFFF))
            a = jnp.maximum(a, 0.2 * a)
            ex = jnp.exp(a - g_v[...])
            expa[sl16] = ex
            for m in range(16):
                coef = _bcast_lane(ex, m)
                e = k * 16 + m
                for q in range(D // 16):
                    sl = pl.ds(q * 16, 16)
                    rows[e, sl] = rows[e, sl] * coef

    pltpu.sync_copy(pal_hbm, pal_v)
    pltpu.sync_copy(g_hbm, g_v)
    start_packed(0, 0)
    start_packed(1, 1)
    wait_packed(0, 0)
    unpack_idx(0)
    start_packed(0, 2)
    start_gathers(0)

    def _zrow(e, carry):
        for q in range(D // 16):
            rows_b[e, pl.ds(q * 16, 16)] = jnp.zeros((16,), _f32)
        return carry

    lax.fori_loop(0, CH, _zrow, 0)

    def _zd(i, carry):
        dbuf[pl.ds(i * 16, 16)] = jnp.zeros((16,), _f32)
        return carry

    lax.fori_loop(0, RPT // 16, _zd, 0)

    for k in range(RPT // CH):
        pltpu.sync_copy(rows_b, out_sp.at[pl.ds(tbase + k * CH, CH)])
    pltpu.sync_copy(dbuf, den_sp.at[pl.ds(tbase, RPT)])
    plsc.subcore_barrier()

    NP = CPT // 2

    def _pair(p, carry):
        ja = 2 * p
        jb = ja + 1

        @pl.when(p > 0)
        def _():
            wait_scatters(1)

        wait_packed(1, jb)
        unpack_idx(1)
        start_packed(1, jnp.minimum(jb + 2, CPT - 1))
        start_gathers(1)
        wait_gathers(0)
        compute(0)
        start_scatters(0)

        wait_gathers(1)
        compute(1)
        start_scatters(1)

        @pl.when(p < NP - 1)
        def _():
            wait_scatters(0)
            wait_packed(0, ja + 2)
            unpack_idx(0)
            start_packed(0, jnp.minimum(ja + 4, CPT - 1))
            start_gathers(0)

        return carry

    lax.fori_loop(0, NP, _pair, 0)
    wait_scatters(0)
    wait_scatters(1)
    wait_packed(0, CPT - 1)
    wait_packed(1, CPT - 1)
    plsc.subcore_barrier()

    obase = cid * NPAD
    for k in range(RPT // CH):
        pltpu.sync_copy(out_sp.at[pl.ds(tbase + k * CH, CH)], rows_a)
        pltpu.sync_copy(rows_a, outp_hbm.at[pl.ds(obase + tbase + k * CH, CH)])
    pltpu.sync_copy(den_sp.at[pl.ds(tbase, RPT)], dbuf)
    pltpu.sync_copy(dbuf, denp_hbm.at[pl.ds(obase + tbase, RPT)])


def _avec(a_s, a_d):
    A = jnp.zeros((D, 8), _f32)
    return A.at[:, 0].set(a_s).at[:, 1].set(a_d)


def _pal(P):
    hi = lax.bitcast_convert_type(P[:, 0].astype(jnp.float16), jnp.uint16)
    lo = lax.bitcast_convert_type(P[:, 1].astype(jnp.float16), jnp.uint16)
    return (hi.astype(jnp.int32) << 16) | lo.astype(jnp.int32)


def _gvec(mx):
    g = jnp.maximum(jnp.max(mx[:, 0]) + jnp.max(mx[:, 1]), 0.0)
    return jnp.full((16,), g, _f32)


def kernel(x, edge_index, edge_weight, W1, a_src1, a_dst1, b1,
           W2, a_src2, a_dst2, b2, W3, a_src3, a_dst3, b3):
    del edge_weight
    xp = jnp.zeros((NPAD, D), _f32).at[:N].set(x)
    loop = jnp.arange(N, dtype=jnp.int32)
    src = jnp.concatenate([edge_index[0].astype(jnp.int32), loop])
    dst = jnp.concatenate([edge_index[1].astype(jnp.int32), loop])
    padidx = N + (jnp.arange(EPAD - EALL, dtype=jnp.int32) % (NPAD - N))
    srcf = jnp.concatenate([src, padidx])
    dstf = jnp.concatenate([dst, padidx])
    sd2 = (srcf | (dstf << 16)).reshape(32, CPT, CH)

    h, P, mx = _tc_head(xp, W1, _avec(a_src1, a_dst1))
    outp, denp = _sc_edge(h, sd2, _pal(P), _gvec(mx))

    h, P, mx = _tc_mid(outp[:NPAD], outp[NPAD:], denp[:NPAD, None],
                       denp[NPAD:, None], b1.reshape(1, D), W2,
                       _avec(a_src2, a_dst2))
    outp, denp = _sc_edge(h, sd2, _pal(P), _gvec(mx))

    h, P, mx = _tc_mid(outp[:NPAD], outp[NPAD:], denp[:NPAD, None],
                       denp[NPAD:, None], b2.reshape(1, D), W3,
                       _avec(a_src3, a_dst3))
    outp, denp = _sc_edge(h, sd2, _pal(P), _gvec(mx))

    out = _tc_tail(outp[:NPAD], outp[NPAD:], denp[:NPAD, None],
                   denp[NPAD:, None], b3.reshape(1, D))
    return out[:N]
